# baseline (speedup 1.0000x reference)
import functools

import numpy as np
import jax
import jax.numpy as jnp
from jax import lax
from jax.experimental import pallas as pl
from jax.experimental.pallas import tpu as pltpu
from jax.experimental.pallas import tpu_sc as plsc

_N = 2048
_R = 4
_L = 8
_H = 256
_C = 16

_TINY = np.float32(np.finfo(np.float32).tiny)
_BI = 128


def _threefry_bits(ks0, ks1, m):
    ks2 = ks0 ^ ks1 ^ np.uint32(0x1BD11BDA)
    ks = (ks0, ks1, ks2)
    rot0 = (13, 15, 26, 6)
    rot1 = (17, 29, 16, 24)
    x0 = jnp.broadcast_to(ks0, m.shape)
    x1 = m + ks1

    def rounds(x0, x1, rs):
        for r in rs:
            x0 = x0 + x1
            x1 = (x1 << np.uint32(r)) | (x1 >> np.uint32(32 - r))
            x1 = x0 ^ x1
        return x0, x1

    sched = ((rot0, 1, 2, 1), (rot1, 2, 0, 2), (rot0, 0, 1, 3),
             (rot1, 1, 2, 4), (rot0, 2, 0, 5))
    for rs, a, b, i in sched:
        x0, x1 = rounds(x0, x1, rs)
        x0 = x0 + ks[a]
        x1 = x1 + ks[b] + np.uint32(i)
    return x0 ^ x1


def _gumbel_from_bits(bits):
    fb = (bits >> np.uint32(9)) | np.uint32(0x3F800000)
    u = jax.lax.bitcast_convert_type(fb, jnp.float32) - np.float32(1.0)
    return -jnp.log(-jnp.log(u + _TINY))


def _logp_kernel(p_ref, o_ref):
    o_ref[:, :] = jnp.log(p_ref[:, :] + np.float32(1e-9))


def _logp(probability):
    return pl.pallas_call(
        _logp_kernel,
        grid=(16,),
        in_specs=[pl.BlockSpec((_N // 16, _N), lambda i: (i, 0))],
        out_specs=pl.BlockSpec((_N // 16, _N), lambda i: (i, 0)),
        out_shape=jax.ShapeDtypeStruct((_N, _N), jnp.float32),
    )(probability)


_NS = 16
_NL = _N // _NS


_NT = _L - 1
_NB = _N // _BI


def _walk_kernel(key_ref, p_ref, out_ref, rows_ref, curs_ref, curv_ref,
                 logp_ref, sem):
    t = pl.program_id(0)
    ib = pl.program_id(1)
    i0 = ib * _BI
    ks0 = key_ref[t, 0]
    ks1 = key_ref[t, 1]
    shp = (_BI, _NS, _NL)
    ii_iota = jax.lax.broadcasted_iota(jnp.int32, shp, 0)
    s_iota = jax.lax.broadcasted_iota(jnp.int32, shp, 1)
    l_iota = jax.lax.broadcasted_iota(jnp.int32, shp, 2)
    j_iota = s_iota * np.int32(_NL) + l_iota
    m_base = ii_iota * np.int32(_N) + j_iota
    j_iota_f = j_iota.astype(jnp.float32)

    @pl.when(jnp.logical_and(t == 0, ib == 0))
    def _init():
        curv_ref[:, :] = jax.lax.broadcasted_iota(jnp.int32, (_R, _N), 1)
        cp = pltpu.make_async_copy(curv_ref, curs_ref, sem)
        cp.start()
        cp.wait()

    @pl.when(t == 0)
    def _fill_logp():
        logp_ref[pl.ds(i0, _BI), :, :] = jnp.log(
            p_ref[:, :, :] + np.float32(1e-9))

    def gather(r, buf):
        for ii in range(_BI):
            row = curs_ref[r, i0 + ii]
            rows_ref[buf, pl.ds(ii, 1), :, :] = logp_ref[pl.ds(row, 1), :, :]

    gather(0, 0)
    for r in range(_R):
        if r + 1 < _R:
            gather(r + 1, (r + 1) % 2)
        moff = np.int32(r * _N * _N) + i0 * np.int32(_N)
        m = (m_base + moff).astype(jnp.uint32)
        g = _gumbel_from_bits(_threefry_bits(ks0, ks1, m))
        val = g + rows_ref[r % 2, :, :, :]
        mx = jnp.max(jnp.max(val, axis=2, keepdims=True), axis=1, keepdims=True)
        cand = jnp.where(val == mx, j_iota_f, np.float32(2**30))
        newcur = jnp.min(jnp.min(cand, axis=2), axis=1).astype(jnp.int32)
        out_ref[0, r, :] = newcur
        curv_ref[r, pl.ds(i0, _BI)] = newcur

    @pl.when(ib == _NB - 1)
    def _pub():
        cp = pltpu.make_async_copy(curv_ref, curs_ref, sem)
        cp.start()
        cp.wait()


def _walk_all(kds, prob3):
    return pl.pallas_call(
        _walk_kernel,
        grid_spec=pltpu.PrefetchScalarGridSpec(
            num_scalar_prefetch=1,
            grid=(_NT, _NB),
            in_specs=[pl.BlockSpec(
                (_BI, _NS, _NL),
                lambda t, ib, *_: (jnp.where(t == 0, ib, 0), 0, 0))],
            out_specs=pl.BlockSpec((1, _R, _BI), lambda t, ib, *_: (t, 0, ib)),
            scratch_shapes=[
                pltpu.VMEM((2, _BI, _NS, _NL), jnp.float32),
                pltpu.SMEM((_R, _N), jnp.int32),
                pltpu.VMEM((_R, _N), jnp.int32),
                pltpu.VMEM((_N, _NS, _NL), jnp.float32),
                pltpu.SemaphoreType.DMA,
            ],
        ),
        out_shape=jax.ShapeDtypeStruct((_NT, _R, _N), jnp.int32),
    )(kds, prob3)


_DIN = 128
_GW = 256
_BN = 256


def _wprep_kernel(w_in_ref, w_top_ref, b_in_ref, b_inv_ref, o_ref):
    o_ref[0:_DIN, :] = jnp.dot(w_in_ref[:, :], w_top_ref[:, :],
                               preferred_element_type=jnp.float32)
    o_ref[_DIN:_DIN + 1, :] = (
        jnp.dot(b_in_ref[:, :], w_top_ref[:, :],
                preferred_element_type=jnp.float32) + b_inv_ref[:, :])


def _wprep(W_in, W_inv_top, b_in, b_inv):
    return pl.pallas_call(
        _wprep_kernel,
        out_shape=jax.ShapeDtypeStruct((_DIN + 1, _H), jnp.float32),
    )(W_in, W_inv_top, b_in.reshape(1, _H), b_inv.reshape(1, _H))


_SC_NC = 2
_SC_NS = 16
_SC_NW = _SC_NC * _SC_NS
_SC_B = _R * _L * _N
_SC_BPW = _SC_B // _SC_NW
_SC_CH = 128


def _sc_gather_body(table_hbm, idx_hbm, out_hbm, idx_v, rows_v, sem):
    wid = lax.axis_index("s") * _SC_NC + lax.axis_index("c")
    base = wid * _SC_BPW
    for c in range(_SC_BPW // _SC_CH):
        off = base + c * _SC_CH
        pltpu.sync_copy(idx_hbm.at[pl.ds(off, _SC_CH)], idx_v)
        pltpu.async_copy(table_hbm.at[idx_v], rows_v, sem).wait()
        pltpu.sync_copy(rows_v, out_hbm.at[pl.ds(off, _SC_CH)])


def _sc_gather(table, idx_flat):
    fn = pl.kernel(
        _sc_gather_body,
        out_type=jax.ShapeDtypeStruct((_SC_B, _GW), jnp.float32),
        mesh=plsc.VectorSubcoreMesh(core_axis_name="c", subcore_axis_name="s"),
        scratch_types=[
            pltpu.VMEM((_SC_CH,), jnp.int32),
            pltpu.VMEM((_SC_CH, _GW), jnp.float32),
            pltpu.SemaphoreType.DMA,
        ],
    )
    return fn(table, idx_flat)


def _layer_kernel(g_ref, wc_ref, w_invh_ref, w_eqn_ref, w_ehn_ref,
                  w_gate_ref, b_gate_ref, w_ch_ref, inv_ref, eq_ref):
    inv_h = jnp.zeros((_BN, _H), dtype=jnp.float32)
    eq_h = [jnp.zeros((_BN, _C), dtype=jnp.float32) for _ in range(3)]
    wch16 = w_ch_ref[0:_C, :]
    wch_last = w_ch_ref[_C:_C + 1, :]
    for idx in range(_L):
        raw = g_ref[0, idx, :, 0:_DIN]
        if idx == 0:
            eq_step = [jnp.zeros((_BN, 1), dtype=jnp.float32) for _ in range(3)]
        else:
            eq_step = [g_ref[0, idx, :, _DIN + d:_DIN + d + 1]
                       - g_ref[0, idx - 1, :, _DIN + d:_DIN + d + 1]
                       for d in range(3)]
        eq_in_norm = jnp.sqrt(eq_step[0] * eq_step[0]
                              + eq_step[1] * eq_step[1]
                              + eq_step[2] * eq_step[2])
        eq_h_norm = jnp.sqrt(eq_h[0] * eq_h[0] + eq_h[1] * eq_h[1]
                             + eq_h[2] * eq_h[2])
        z = (jnp.dot(raw, wc_ref[0:_DIN, :], preferred_element_type=jnp.float32)
             + jnp.dot(inv_h, w_invh_ref[:, :], preferred_element_type=jnp.float32)
             + eq_in_norm * w_eqn_ref[:, :]
             + jnp.dot(eq_h_norm, w_ehn_ref[:, :], preferred_element_type=jnp.float32)
             + wc_ref[_DIN:_DIN + 1, :])
        inv_h = jnp.tanh(z)
        gates = jax.nn.sigmoid(
            jnp.dot(inv_h, w_gate_ref[:, :], preferred_element_type=jnp.float32)
            + b_gate_ref[:, :])
        eq_h = [(jnp.dot(eq_h[d], wch16, preferred_element_type=jnp.float32)
                 + eq_step[d] * wch_last) * gates for d in range(3)]
        inv_ref[0, idx, :, :] = inv_h
        for d in range(3):
            eq_ref[0, idx, :, d, :] = eq_h[d]


def _layers(g, wc, W_inv, W_gate, b_gate, W_ch):
    w_invh = W_inv[_H:2 * _H, :]
    w_eqn = W_inv[2 * _H:2 * _H + 1, :]
    w_ehn = W_inv[2 * _H + 1:, :]
    nb = _N // _BN
    inv_traj, eq_tmp = pl.pallas_call(
        _layer_kernel,
        grid=(_R, nb),
        in_specs=[
            pl.BlockSpec((1, _L, _BN, _GW), lambda r, ib: (r, 0, ib, 0)),
            pl.BlockSpec((_DIN + 1, _H), lambda r, ib: (0, 0)),
            pl.BlockSpec((_H, _H), lambda r, ib: (0, 0)),
            pl.BlockSpec((1, _H), lambda r, ib: (0, 0)),
            pl.BlockSpec((_C, _H), lambda r, ib: (0, 0)),
            pl.BlockSpec((_H, _C), lambda r, ib: (0, 0)),
            pl.BlockSpec((1, _C), lambda r, ib: (0, 0)),
            pl.BlockSpec((_C + 1, _C), lambda r, ib: (0, 0)),
        ],
        out_specs=[
            pl.BlockSpec((1, _L, _BN, _H), lambda r, ib: (r, 0, ib, 0)),
            pl.BlockSpec((1, _L, _BN, 3, _C), lambda r, ib: (r, 0, ib, 0, 0)),
        ],
        out_shape=[
            jax.ShapeDtypeStruct((_R, _L, _N, _H), jnp.float32),
            jax.ShapeDtypeStruct((_R, _L, _N, 3, _C), jnp.float32),
        ],
        compiler_params=pltpu.CompilerParams(
            dimension_semantics=("parallel", "parallel")),
    )(g, wc, w_invh, w_eqn, w_ehn, W_gate, b_gate.reshape(1, _C), W_ch)
    return inv_traj, eq_tmp


def kernel(probability, invariant_input, equivariant_input, W_in, b_in,
           W_inv, b_inv, W_gate, b_gate, W_ch):
    base = jax.random.key(42)
    kds = jnp.stack(
        [jax.random.key_data(jax.random.fold_in(base, t)).astype(jnp.uint32)
         for t in range(_NT)])
    cur0 = jnp.broadcast_to(jnp.arange(_N, dtype=jnp.int32), (1, _R, _N))
    steps_arr = _walk_all(kds, probability.reshape(_N, _NS, _NL))
    walks = jnp.swapaxes(
        jnp.concatenate([steps_arr[::-1], cur0], axis=0), 0, 1)
    table = jnp.concatenate(
        [invariant_input, equivariant_input,
         jnp.zeros((_N, _GW - _DIN - 3), jnp.float32)], axis=1)
    g = _sc_gather(table, walks.reshape(-1)).reshape(_R, _L, _N, _GW)
    wc = _wprep(W_in, W_inv[0:_H, :], b_in, b_inv)
    return _layers(g, wc, W_inv, W_gate, b_gate, W_ch)


# speedup vs baseline: 1.0467x; 1.0467x over previous
import functools

import numpy as np
import jax
import jax.numpy as jnp
from jax import lax
from jax.experimental import pallas as pl
from jax.experimental.pallas import tpu as pltpu
from jax.experimental.pallas import tpu_sc as plsc

_N = 2048
_R = 4
_L = 8
_H = 256
_C = 16

_TINY = np.float32(np.finfo(np.float32).tiny)
_BI = 128


def _threefry_bits(ks0, ks1, m):
    ks2 = ks0 ^ ks1 ^ np.uint32(0x1BD11BDA)
    ks = (ks0, ks1, ks2)
    rot0 = (13, 15, 26, 6)
    rot1 = (17, 29, 16, 24)
    x0 = jnp.broadcast_to(ks0, m.shape)
    x1 = m + ks1

    def rounds(x0, x1, rs):
        for r in rs:
            x0 = x0 + x1
            x1 = (x1 << np.uint32(r)) | (x1 >> np.uint32(32 - r))
            x1 = x0 ^ x1
        return x0, x1

    sched = ((rot0, 1, 2, 1), (rot1, 2, 0, 2), (rot0, 0, 1, 3),
             (rot1, 1, 2, 4), (rot0, 2, 0, 5))
    for rs, a, b, i in sched:
        x0, x1 = rounds(x0, x1, rs)
        x0 = x0 + ks[a]
        x1 = x1 + ks[b] + np.uint32(i)
    return x0 ^ x1


def _gumbel_from_bits(bits):
    fb = (bits >> np.uint32(9)) | np.uint32(0x3F800000)
    u = jax.lax.bitcast_convert_type(fb, jnp.float32) - np.float32(1.0)
    return -jnp.log(-jnp.log(u + _TINY))


def _logp_kernel(p_ref, o_ref):
    o_ref[:, :] = jnp.log(p_ref[:, :] + np.float32(1e-9))


def _logp(probability):
    return pl.pallas_call(
        _logp_kernel,
        grid=(16,),
        in_specs=[pl.BlockSpec((_N // 16, _N), lambda i: (i, 0))],
        out_specs=pl.BlockSpec((_N // 16, _N), lambda i: (i, 0)),
        out_shape=jax.ShapeDtypeStruct((_N, _N), jnp.float32),
    )(probability)


_NS = 16
_NL = _N // _NS


_NT = _L - 1
_NB = _N // _BI


def _walk_kernel(key_ref, p_ref, out_ref, rows_ref, curs_ref, curv_ref,
                 logp_ref, sem):
    t = pl.program_id(0)
    ib = pl.program_id(1)
    i0 = ib * _BI
    ks0 = key_ref[t, 0]
    ks1 = key_ref[t, 1]
    shp = (_BI, _NS, _NL)
    ii_iota = jax.lax.broadcasted_iota(jnp.int32, shp, 0)
    s_iota = jax.lax.broadcasted_iota(jnp.int32, shp, 1)
    l_iota = jax.lax.broadcasted_iota(jnp.int32, shp, 2)
    j_iota = s_iota * np.int32(_NL) + l_iota
    m_base = ii_iota * np.int32(_N) + j_iota
    j_iota_f = j_iota.astype(jnp.float32)

    @pl.when(jnp.logical_and(t == 0, ib == 0))
    def _init():
        curv_ref[:, :] = jax.lax.broadcasted_iota(jnp.int32, (_R, _N), 1)
        cp = pltpu.make_async_copy(curv_ref, curs_ref, sem)
        cp.start()
        cp.wait()

    @pl.when(t == 0)
    def _fill_logp():
        logp_ref[pl.ds(i0, _BI), :, :] = jnp.log(
            p_ref[:, :, :] + np.float32(1e-9))

    def gather(r, buf):
        for ii in range(_BI):
            row = curs_ref[r, i0 + ii]
            rows_ref[buf, pl.ds(ii, 1), :, :] = logp_ref[pl.ds(row, 1), :, :]

    gather(0, 0)
    for r in range(_R):
        if r + 1 < _R:
            gather(r + 1, (r + 1) % 2)
        moff = np.int32(r * _N * _N) + i0 * np.int32(_N)
        m = (m_base + moff).astype(jnp.uint32)
        g = _gumbel_from_bits(_threefry_bits(ks0, ks1, m))
        val = g + rows_ref[r % 2, :, :, :]
        mx = jnp.max(jnp.max(val, axis=2, keepdims=True), axis=1, keepdims=True)
        cand = jnp.where(val == mx, j_iota_f, np.float32(2**30))
        newcur = jnp.min(jnp.min(cand, axis=2), axis=1).astype(jnp.int32)
        out_ref[0, r, :] = newcur
        curv_ref[r, pl.ds(i0, _BI)] = newcur

    @pl.when(ib == _NB - 1)
    def _pub():
        cp = pltpu.make_async_copy(curv_ref, curs_ref, sem)
        cp.start()
        cp.wait()


def _walk_all(kds, prob3):
    return pl.pallas_call(
        _walk_kernel,
        grid_spec=pltpu.PrefetchScalarGridSpec(
            num_scalar_prefetch=1,
            grid=(_NT, _NB),
            in_specs=[pl.BlockSpec(
                (_BI, _NS, _NL),
                lambda t, ib, *_: (jnp.where(t == 0, ib, 0), 0, 0))],
            out_specs=pl.BlockSpec((1, _R, _BI), lambda t, ib, *_: (t, 0, ib)),
            scratch_shapes=[
                pltpu.VMEM((2, _BI, _NS, _NL), jnp.float32),
                pltpu.SMEM((_R, _N), jnp.int32),
                pltpu.VMEM((_R, _N), jnp.int32),
                pltpu.VMEM((_N, _NS, _NL), jnp.float32),
                pltpu.SemaphoreType.DMA,
            ],
        ),
        out_shape=jax.ShapeDtypeStruct((_NT, _R, _N), jnp.int32),
    )(kds, prob3)


_DIN = 128
_GW = 256
_BN = 256


def _wprep_kernel(w_in_ref, w_top_ref, b_in_ref, b_inv_ref, o_ref):
    o_ref[0:_DIN, :] = jnp.dot(w_in_ref[:, :], w_top_ref[:, :],
                               preferred_element_type=jnp.float32)
    o_ref[_DIN:_DIN + 1, :] = (
        jnp.dot(b_in_ref[:, :], w_top_ref[:, :],
                preferred_element_type=jnp.float32) + b_inv_ref[:, :])


def _wprep(W_in, W_inv_top, b_in, b_inv):
    return pl.pallas_call(
        _wprep_kernel,
        out_shape=jax.ShapeDtypeStruct((_DIN + 1, _H), jnp.float32),
    )(W_in, W_inv_top, b_in.reshape(1, _H), b_inv.reshape(1, _H))


_SC_NC = 2
_SC_NS = 16
_SC_NW = _SC_NC * _SC_NS
_SC_B = _R * _L * _N
_SC_BPW = _SC_B // _SC_NW
_SC_CH = 128


def _sc_gather_body(table_hbm, idx_hbm, out_hbm, idx_v, rows_v, sem):
    wid = lax.axis_index("s") * _SC_NC + lax.axis_index("c")
    base = wid * _SC_BPW
    for c in range(_SC_BPW // _SC_CH):
        off = base + c * _SC_CH
        pltpu.sync_copy(idx_hbm.at[pl.ds(off, _SC_CH)], idx_v)
        pltpu.async_copy(table_hbm.at[idx_v], rows_v, sem).wait()
        pltpu.sync_copy(rows_v, out_hbm.at[pl.ds(off, _SC_CH)])


def _sc_gather(table, idx_flat):
    fn = pl.kernel(
        _sc_gather_body,
        out_type=jax.ShapeDtypeStruct((_SC_B, _GW), jnp.float32),
        mesh=plsc.VectorSubcoreMesh(core_axis_name="c", subcore_axis_name="s"),
        scratch_types=[
            pltpu.VMEM((_SC_CH,), jnp.int32),
            pltpu.VMEM((_SC_CH, _GW), jnp.float32),
            pltpu.SemaphoreType.DMA,
        ],
    )
    return fn(table, idx_flat)


def _layer_kernel(g_ref, wc_ref, w_invh_ref, w_eqn_ref, w_ehn_ref,
                  w_gate_ref, b_gate_ref, w_ch_ref, inv_ref, eq_ref):
    inv_h = jnp.zeros((_BN, _H), dtype=jnp.float32)
    eq_h = [jnp.zeros((_BN, _C), dtype=jnp.float32) for _ in range(3)]
    wch16 = w_ch_ref[0:_C, :]
    wch_last = w_ch_ref[_C:_C + 1, :]
    for idx in range(_L):
        raw = g_ref[0, idx, :, 0:_DIN]
        if idx == 0:
            eq_step = [jnp.zeros((_BN, 1), dtype=jnp.float32) for _ in range(3)]
        else:
            eq_step = [g_ref[0, idx, :, _DIN + d:_DIN + d + 1]
                       - g_ref[0, idx - 1, :, _DIN + d:_DIN + d + 1]
                       for d in range(3)]
        eq_in_norm = jnp.sqrt(eq_step[0] * eq_step[0]
                              + eq_step[1] * eq_step[1]
                              + eq_step[2] * eq_step[2])
        eq_h_norm = jnp.sqrt(eq_h[0] * eq_h[0] + eq_h[1] * eq_h[1]
                             + eq_h[2] * eq_h[2])
        z = (jnp.dot(raw, wc_ref[0:_DIN, :], preferred_element_type=jnp.float32)
             + jnp.dot(inv_h, w_invh_ref[:, :], preferred_element_type=jnp.float32)
             + eq_in_norm * w_eqn_ref[:, :]
             + jnp.dot(eq_h_norm, w_ehn_ref[:, :], preferred_element_type=jnp.float32)
             + wc_ref[_DIN:_DIN + 1, :])
        inv_h = jnp.tanh(z)
        gates = jax.nn.sigmoid(
            jnp.dot(inv_h, w_gate_ref[:, :], preferred_element_type=jnp.float32)
            + b_gate_ref[:, :])
        eq_h = [(jnp.dot(eq_h[d], wch16, preferred_element_type=jnp.float32)
                 + eq_step[d] * wch_last) * gates for d in range(3)]
        inv_ref[0, idx, :, :] = inv_h
        for d in range(3):
            eq_ref[0, idx, d, :, :] = eq_h[d]


def _layers(g, wc, W_inv, W_gate, b_gate, W_ch):
    w_invh = W_inv[_H:2 * _H, :]
    w_eqn = W_inv[2 * _H:2 * _H + 1, :]
    w_ehn = W_inv[2 * _H + 1:, :]
    nb = _N // _BN
    inv_traj, eq_tmp = pl.pallas_call(
        _layer_kernel,
        grid=(_R, nb),
        in_specs=[
            pl.BlockSpec((1, _L, _BN, _GW), lambda r, ib: (r, 0, ib, 0)),
            pl.BlockSpec((_DIN + 1, _H), lambda r, ib: (0, 0)),
            pl.BlockSpec((_H, _H), lambda r, ib: (0, 0)),
            pl.BlockSpec((1, _H), lambda r, ib: (0, 0)),
            pl.BlockSpec((_C, _H), lambda r, ib: (0, 0)),
            pl.BlockSpec((_H, _C), lambda r, ib: (0, 0)),
            pl.BlockSpec((1, _C), lambda r, ib: (0, 0)),
            pl.BlockSpec((_C + 1, _C), lambda r, ib: (0, 0)),
        ],
        out_specs=[
            pl.BlockSpec((1, _L, _BN, _H), lambda r, ib: (r, 0, ib, 0)),
            pl.BlockSpec((1, _L, 3, _BN, _C), lambda r, ib: (r, 0, 0, ib, 0)),
        ],
        out_shape=[
            jax.ShapeDtypeStruct((_R, _L, _N, _H), jnp.float32),
            jax.ShapeDtypeStruct((_R, _L, 3, _N, _C), jnp.float32),
        ],
        compiler_params=pltpu.CompilerParams(
            dimension_semantics=("parallel", "parallel")),
    )(g, wc, w_invh, w_eqn, w_ehn, W_gate, b_gate.reshape(1, _C), W_ch)
    return inv_traj, jnp.moveaxis(eq_tmp, 2, 3)


def kernel(probability, invariant_input, equivariant_input, W_in, b_in,
           W_inv, b_inv, W_gate, b_gate, W_ch):
    base = jax.random.key(42)
    kds = jnp.stack(
        [jax.random.key_data(jax.random.fold_in(base, t)).astype(jnp.uint32)
         for t in range(_NT)])
    cur0 = jnp.broadcast_to(jnp.arange(_N, dtype=jnp.int32), (1, _R, _N))
    steps_arr = _walk_all(kds, probability.reshape(_N, _NS, _NL))
    walks = jnp.swapaxes(
        jnp.concatenate([steps_arr[::-1], cur0], axis=0), 0, 1)
    table = jnp.concatenate(
        [invariant_input, equivariant_input,
         jnp.zeros((_N, _GW - _DIN - 3), jnp.float32)], axis=1)
    g = _sc_gather(table, walks.reshape(-1)).reshape(_R, _L, _N, _GW)
    wc = _wprep(W_in, W_inv[0:_H, :], b_in, b_inv)
    return _layers(g, wc, W_inv, W_gate, b_gate, W_ch)


# speedup vs baseline: 1.0474x; 1.0006x over previous
import numpy as np
import jax
import jax.numpy as jnp
from jax import lax
from jax.experimental import pallas as pl
from jax.experimental.pallas import tpu as pltpu
from jax.experimental.pallas import tpu_sc as plsc

_N = 2048
_R = 4
_L = 8
_H = 256
_C = 16

_TINY = np.float32(np.finfo(np.float32).tiny)
_BI = 128


def _threefry_bits(ks0, ks1, m):
    ks2 = ks0 ^ ks1 ^ np.uint32(0x1BD11BDA)
    ks = (ks0, ks1, ks2)
    rot0 = (13, 15, 26, 6)
    rot1 = (17, 29, 16, 24)
    x0 = jnp.broadcast_to(ks0, m.shape)
    x1 = m + ks1

    def rounds(x0, x1, rs):
        for r in rs:
            x0 = x0 + x1
            x1 = (x1 << np.uint32(r)) | (x1 >> np.uint32(32 - r))
            x1 = x0 ^ x1
        return x0, x1

    sched = ((rot0, 1, 2, 1), (rot1, 2, 0, 2), (rot0, 0, 1, 3),
             (rot1, 1, 2, 4), (rot0, 2, 0, 5))
    for rs, a, b, i in sched:
        x0, x1 = rounds(x0, x1, rs)
        x0 = x0 + ks[a]
        x1 = x1 + ks[b] + np.uint32(i)
    return x0 ^ x1


def _gumbel_from_bits(bits):
    fb = (bits >> np.uint32(9)) | np.uint32(0x3F800000)
    u = jax.lax.bitcast_convert_type(fb, jnp.float32) - np.float32(1.0)
    return -jnp.log(-jnp.log(u + _TINY))


_NS = 16
_NL = _N // _NS


_NT = _L - 1
_NB = _N // _BI


def _walk_kernel(key_ref, p_ref, out_ref, rows_ref, curs_ref, curv_ref,
                 logp_ref, sem):
    t = pl.program_id(0)
    ib = pl.program_id(1)
    i0 = ib * _BI
    ks0 = key_ref[t, 0]
    ks1 = key_ref[t, 1]
    shp = (_BI, _NS, _NL)
    ii_iota = jax.lax.broadcasted_iota(jnp.int32, shp, 0)
    s_iota = jax.lax.broadcasted_iota(jnp.int32, shp, 1)
    l_iota = jax.lax.broadcasted_iota(jnp.int32, shp, 2)
    j_iota = s_iota * np.int32(_NL) + l_iota
    m_base = ii_iota * np.int32(_N) + j_iota
    j_iota_f = j_iota.astype(jnp.float32)

    @pl.when(jnp.logical_and(t == 0, ib == 0))
    def _init():
        curv_ref[:, :] = jax.lax.broadcasted_iota(jnp.int32, (_R, _N), 1)
        cp = pltpu.make_async_copy(curv_ref, curs_ref, sem)
        cp.start()
        cp.wait()

    @pl.when(t == 0)
    def _fill_logp():
        logp_ref[pl.ds(i0, _BI), :, :] = jnp.log(
            p_ref[:, :, :] + np.float32(1e-9))

    def gather(r, buf):
        for ii in range(_BI):
            row = curs_ref[r, i0 + ii]
            rows_ref[buf, pl.ds(ii, 1), :, :] = logp_ref[pl.ds(row, 1), :, :]

    gather(0, 0)
    for r in range(_R):
        if r + 1 < _R:
            gather(r + 1, (r + 1) % 2)
        moff = np.int32(r * _N * _N) + i0 * np.int32(_N)
        m = (m_base + moff).astype(jnp.uint32)
        g = _gumbel_from_bits(_threefry_bits(ks0, ks1, m))
        val = g + rows_ref[r % 2, :, :, :]
        mx = jnp.max(jnp.max(val, axis=2, keepdims=True), axis=1, keepdims=True)
        cand = jnp.where(val == mx, j_iota_f, np.float32(2**30))
        newcur = jnp.min(jnp.min(cand, axis=2), axis=1).astype(jnp.int32)
        out_ref[0, r, :] = newcur
        curv_ref[r, pl.ds(i0, _BI)] = newcur

    @pl.when(ib == _NB - 1)
    def _pub():
        cp = pltpu.make_async_copy(curv_ref, curs_ref, sem)
        cp.start()
        cp.wait()


def _walk_all(kds, prob3):
    return pl.pallas_call(
        _walk_kernel,
        grid_spec=pltpu.PrefetchScalarGridSpec(
            num_scalar_prefetch=1,
            grid=(_NT, _NB),
            in_specs=[pl.BlockSpec(
                (_BI, _NS, _NL),
                lambda t, ib, *_: (jnp.where(t == 0, ib, 0), 0, 0))],
            out_specs=pl.BlockSpec((1, _R, _BI), lambda t, ib, *_: (t, 0, ib)),
            scratch_shapes=[
                pltpu.VMEM((2, _BI, _NS, _NL), jnp.float32),
                pltpu.SMEM((_R, _N), jnp.int32),
                pltpu.VMEM((_R, _N), jnp.int32),
                pltpu.VMEM((_N, _NS, _NL), jnp.float32),
                pltpu.SemaphoreType.DMA,
            ],
        ),
        out_shape=jax.ShapeDtypeStruct((_NT, _R, _N), jnp.int32),
    )(kds, prob3)


_DIN = 128
_GW = 256
_BN = 256


def _wprep_kernel(w_in_ref, w_top_ref, b_in_ref, b_inv_ref, o_ref):
    o_ref[0:_DIN, :] = jnp.dot(w_in_ref[:, :], w_top_ref[:, :],
                               preferred_element_type=jnp.float32)
    o_ref[_DIN:_DIN + 1, :] = (
        jnp.dot(b_in_ref[:, :], w_top_ref[:, :],
                preferred_element_type=jnp.float32) + b_inv_ref[:, :])


def _wprep(W_in, W_inv_top, b_in, b_inv):
    return pl.pallas_call(
        _wprep_kernel,
        out_shape=jax.ShapeDtypeStruct((_DIN + 1, _H), jnp.float32),
    )(W_in, W_inv_top, b_in.reshape(1, _H), b_inv.reshape(1, _H))


_SC_NC = 2
_SC_NS = 16
_SC_NW = _SC_NC * _SC_NS
_SC_B = _R * _L * _N
_SC_BPW = _SC_B // _SC_NW
_SC_CH = 128


def _sc_gather_body(table_hbm, idx_hbm, out_hbm, idx_v, rows_v, sem):
    wid = lax.axis_index("s") * _SC_NC + lax.axis_index("c")
    base = wid * _SC_BPW
    for c in range(_SC_BPW // _SC_CH):
        off = base + c * _SC_CH
        pltpu.sync_copy(idx_hbm.at[pl.ds(off, _SC_CH)], idx_v)
        pltpu.async_copy(table_hbm.at[idx_v], rows_v, sem).wait()
        pltpu.sync_copy(rows_v, out_hbm.at[pl.ds(off, _SC_CH)])


def _sc_gather(table, idx_flat):
    fn = pl.kernel(
        _sc_gather_body,
        out_type=jax.ShapeDtypeStruct((_SC_B, _GW), jnp.float32),
        mesh=plsc.VectorSubcoreMesh(core_axis_name="c", subcore_axis_name="s"),
        scratch_types=[
            pltpu.VMEM((_SC_CH,), jnp.int32),
            pltpu.VMEM((_SC_CH, _GW), jnp.float32),
            pltpu.SemaphoreType.DMA,
        ],
    )
    return fn(table, idx_flat)


def _layer_kernel(g_ref, wc_ref, w_invh_ref, w_eqn_ref, w_ehn_ref,
                  w_gate_ref, b_gate_ref, w_ch_ref, inv_ref, eq_ref):
    inv_h = jnp.zeros((_BN, _H), dtype=jnp.float32)
    eq_h = [jnp.zeros((_BN, _C), dtype=jnp.float32) for _ in range(3)]
    wch16 = w_ch_ref[0:_C, :]
    wch_last = w_ch_ref[_C:_C + 1, :]
    for idx in range(_L):
        raw = g_ref[0, idx, :, 0:_DIN]
        if idx == 0:
            eq_step = [jnp.zeros((_BN, 1), dtype=jnp.float32) for _ in range(3)]
        else:
            eq_step = [g_ref[0, idx, :, _DIN + d:_DIN + d + 1]
                       - g_ref[0, idx - 1, :, _DIN + d:_DIN + d + 1]
                       for d in range(3)]
        eq_in_norm = jnp.sqrt(eq_step[0] * eq_step[0]
                              + eq_step[1] * eq_step[1]
                              + eq_step[2] * eq_step[2])
        eq_h_norm = jnp.sqrt(eq_h[0] * eq_h[0] + eq_h[1] * eq_h[1]
                             + eq_h[2] * eq_h[2])
        z = (jnp.dot(raw, wc_ref[0:_DIN, :], preferred_element_type=jnp.float32)
             + jnp.dot(inv_h, w_invh_ref[:, :], preferred_element_type=jnp.float32)
             + eq_in_norm * w_eqn_ref[:, :]
             + jnp.dot(eq_h_norm, w_ehn_ref[:, :], preferred_element_type=jnp.float32)
             + wc_ref[_DIN:_DIN + 1, :])
        inv_h = jnp.tanh(z)
        gates = jax.nn.sigmoid(
            jnp.dot(inv_h, w_gate_ref[:, :], preferred_element_type=jnp.float32)
            + b_gate_ref[:, :])
        eq_h = [(jnp.dot(eq_h[d], wch16, preferred_element_type=jnp.float32)
                 + eq_step[d] * wch_last) * gates for d in range(3)]
        inv_ref[0, idx, :, :] = inv_h
        for d in range(3):
            eq_ref[0, idx, d, :, :] = eq_h[d]


def _layers(g, wc, W_inv, W_gate, b_gate, W_ch):
    w_invh = W_inv[_H:2 * _H, :]
    w_eqn = W_inv[2 * _H:2 * _H + 1, :]
    w_ehn = W_inv[2 * _H + 1:, :]
    nb = _N // _BN
    inv_traj, eq_tmp = pl.pallas_call(
        _layer_kernel,
        grid=(_R, nb),
        in_specs=[
            pl.BlockSpec((1, _L, _BN, _GW), lambda r, ib: (r, 0, ib, 0)),
            pl.BlockSpec((_DIN + 1, _H), lambda r, ib: (0, 0)),
            pl.BlockSpec((_H, _H), lambda r, ib: (0, 0)),
            pl.BlockSpec((1, _H), lambda r, ib: (0, 0)),
            pl.BlockSpec((_C, _H), lambda r, ib: (0, 0)),
            pl.BlockSpec((_H, _C), lambda r, ib: (0, 0)),
            pl.BlockSpec((1, _C), lambda r, ib: (0, 0)),
            pl.BlockSpec((_C + 1, _C), lambda r, ib: (0, 0)),
        ],
        out_specs=[
            pl.BlockSpec((1, _L, _BN, _H), lambda r, ib: (r, 0, ib, 0)),
            pl.BlockSpec((1, _L, 3, _BN, _C), lambda r, ib: (r, 0, 0, ib, 0)),
        ],
        out_shape=[
            jax.ShapeDtypeStruct((_R, _L, _N, _H), jnp.float32),
            jax.ShapeDtypeStruct((_R, _L, 3, _N, _C), jnp.float32),
        ],
        compiler_params=pltpu.CompilerParams(
            dimension_semantics=("parallel", "parallel")),
    )(g, wc, w_invh, w_eqn, w_ehn, W_gate, b_gate.reshape(1, _C), W_ch)
    return inv_traj, jnp.moveaxis(eq_tmp, 2, 3)


def kernel(probability, invariant_input, equivariant_input, W_in, b_in,
           W_inv, b_inv, W_gate, b_gate, W_ch):
    base = jax.random.key(42)
    kds = jnp.stack(
        [jax.random.key_data(jax.random.fold_in(base, t)).astype(jnp.uint32)
         for t in range(_NT)])
    cur0 = jnp.broadcast_to(jnp.arange(_N, dtype=jnp.int32), (1, _R, _N))
    steps_arr = _walk_all(kds, probability.reshape(_N, _NS, _NL))
    walks = jnp.swapaxes(
        jnp.concatenate([steps_arr[::-1], cur0], axis=0), 0, 1)
    table = jnp.concatenate(
        [invariant_input, equivariant_input,
         jnp.zeros((_N, _GW - _DIN - 3), jnp.float32)], axis=1)
    g = _sc_gather(table, walks.reshape(-1)).reshape(_R, _L, _N, _GW)
    wc = _wprep(W_in, W_inv[0:_H, :], b_in, b_inv)
    return _layers(g, wc, W_inv, W_gate, b_gate, W_ch)
